# baseline (speedup 1.0000x reference)
import functools

import jax
import jax.numpy as jnp
from jax import lax
from jax.experimental import pallas as pl
from jax.experimental.pallas import tpu as pltpu
from jax.experimental.pallas import tpu_sc as plsc

_V = 100000
_D = 300
_B = 1024
_NC = 2
_NS = 16
_NW = _NC * _NS
_B_PER_W = _B // _NW

_TV = 2048
_GRID = (_V + _TV - 1) // _TV


_DP = 304


def _sc_gather(emb_table_p, idx):
    mesh = plsc.VectorSubcoreMesh(core_axis_name="c", subcore_axis_name="s")

    @functools.partial(
        pl.kernel,
        mesh=mesh,
        out_type=jax.ShapeDtypeStruct((_B, _DP), jnp.float32),
        scratch_types=[
            pltpu.VMEM((_B_PER_W,), jnp.int32),
            pltpu.VMEM((_B_PER_W, _DP), jnp.float32),
            pltpu.SemaphoreType.DMA,
        ],
        compiler_params=pltpu.CompilerParams(use_tc_tiling_on_sc=False),
    )
    def k(table_hbm, idx_hbm, out_hbm, idx_v, rows_v, sem):
        wid = lax.axis_index("s") * _NC + lax.axis_index("c")
        base = wid * _B_PER_W
        pltpu.sync_copy(idx_hbm.at[pl.ds(base, _B_PER_W)], idx_v)
        pltpu.async_copy(table_hbm.at[idx_v], rows_v, sem).wait()
        pltpu.sync_copy(rows_v, out_hbm.at[pl.ds(base, _B_PER_W)])

    return k(emb_table_p, idx)


def _matmul_body(x_ref, w_ref, b_ref, o_ref):
    x = x_ref[...][:, :_D]
    norm = jnp.sqrt(jnp.sum(x * x, axis=1, keepdims=True))
    scale = jnp.where(norm > 1.0, 1.0 / (norm + 1e-7), 1.0)
    xs = x * scale
    o_ref[...] = lax.dot_general(
        xs, w_ref[...], (((1,), (1,)), ((), ())),
        preferred_element_type=jnp.float32,
    ) + b_ref[...]


def _tc_project(x, W, b2):
    return pl.pallas_call(
        _matmul_body,
        grid=(_GRID,),
        in_specs=[
            pl.BlockSpec((_B, _DP), lambda j: (0, 0)),
            pl.BlockSpec((_TV, _D), lambda j: (j, 0)),
            pl.BlockSpec((1, _TV), lambda j: (0, j)),
        ],
        out_specs=pl.BlockSpec((_B, _TV), lambda j: (0, j)),
        out_shape=jax.ShapeDtypeStruct((_B, _V), jnp.float32),
        compiler_params=pltpu.CompilerParams(
            dimension_semantics=("arbitrary",),
        ),
    )(x, W, b2)


def kernel(inputs_, emb_table, W, b):
    emb_p = jnp.pad(emb_table, ((0, 0), (0, _DP - _D)))
    x = _sc_gather(emb_p, inputs_.astype(jnp.int32))
    return _tc_project(x, W, b.reshape(1, _V))
